# Initial kernel scaffold; baseline (speedup 1.0000x reference)
import jax
import jax.numpy as jnp
from jax.experimental import pallas as pl


def kernel(x, edge_index, batch, W1, b1, g1, bt1, pw1, W2, b2, g2, bt2, pw2, Wf, bf, Wo, bo):
    raise NotImplementedError("write your pallas kernel here")


# speedup vs baseline: 6.8487x; 6.8487x over previous
import functools

import jax
import jax.numpy as jnp
from jax import lax
from jax.experimental import pallas as pl
from jax.experimental.pallas import tpu as pltpu
from jax.experimental.pallas import tpu_sc as plsc

_N = 10000
_ND = 112
_NR = _N + _ND
_E = 160000
_EPAD = 163840
_B = 128
_NBM = 40
_NBR = 80
_RPT = _NR // 16
_NB = 1000
_GN = _N // _NB
_EPS = 1e-5

_mesh = plsc.VectorSubcoreMesh(core_axis_name="c", subcore_axis_name="s")
_sc_params = pltpu.CompilerParams(needs_layout_passes=False)


def _sc_deg(keep_ext, src32, dst32, zeros_nr):
  @functools.partial(
      pl.kernel,
      out_type=jax.ShapeDtypeStruct((32, 1, _NR), jnp.float32),
      mesh=_mesh,
      compiler_params=_sc_params,
      scratch_types=[
          pltpu.VMEM((_NR,), jnp.float32),
          pltpu.VMEM((_NBM, _B), jnp.int32),
          pltpu.VMEM((_NBM, _B), jnp.int32),
          pltpu.VMEM((1, _NR), jnp.float32),
      ],
  )
  def body(keep_h, src_h, dst_h, z_h, deg_h, kbuf, sbuf, dbuf, degbuf):
    c = lax.axis_index("c")
    s = lax.axis_index("s")
    w = s * 2 + c
    pltpu.sync_copy(keep_h, kbuf)
    pltpu.sync_copy(src_h.at[w], sbuf)
    pltpu.sync_copy(dst_h.at[w], dbuf)
    pltpu.sync_copy(z_h, degbuf)
    zi = jnp.zeros((16,), jnp.int32)

    def batch(b, carry):
      for j in range(_B // 16):
        sv = sbuf[b, pl.ds(j * 16, 16)]
        dv = dbuf[b, pl.ds(j * 16, 16)]
        ks = plsc.load_gather(kbuf, [sv])
        kd = plsc.load_gather(kbuf, [dv])
        plsc.addupdate_scatter(degbuf, [zi, dv], ks * kd)
      return carry

    lax.fori_loop(0, _NBM, batch, 0)
    pltpu.sync_copy(degbuf, deg_h.at[w])

  return body(keep_ext, src32, dst32, zeros_nr)


def _sc_rows(y, src16, dst16, zeros128, nchunks):
  cp = nchunks // 2

  @functools.partial(
      pl.kernel,
      out_type=jax.ShapeDtypeStruct((nchunks, _NR, _B), jnp.float32),
      mesh=_mesh,
      compiler_params=_sc_params,
      scratch_types=[
          pltpu.VMEM((_NBR, _B), jnp.int32),
          pltpu.VMEM((_NBR, _B), jnp.int32),
          pltpu.VMEM((_B, _B), jnp.float32),
          pltpu.VMEM_SHARED((_NR, _B), jnp.float32),
          pltpu.SemaphoreType.DMA,
      ],
  )
  def body(y_h, src_h, dst_h, z_h, out_h, sbuf, dbuf, rbuf, acc, sem):
    c = lax.axis_index("c")
    s = lax.axis_index("s")
    pltpu.sync_copy(src_h.at[s], sbuf)
    pltpu.sync_copy(dst_h.at[s], dbuf)
    for ci in range(cp):
      ch = c * cp + ci
      pltpu.sync_copy(z_h, acc.at[pl.ds(s * _RPT, _RPT)])
      plsc.subcore_barrier()

      def batch(b, carry):
        pltpu.async_copy(y_h.at[ch].at[sbuf.at[b]], rbuf, sem).wait()
        pltpu.sync_copy(rbuf, acc.at[dbuf.at[b]], add=True)
        return carry

      lax.fori_loop(0, _NBR, batch, 0)
      plsc.subcore_barrier()
      pltpu.sync_copy(acc.at[pl.ds(s * _RPT, _RPT)],
                      out_h.at[ch, pl.ds(s * _RPT, _RPT)])
      plsc.subcore_barrier()

  return body(y, src16, dst16, zeros128)


def _deg_to_dinv(deg_blk):
  d = jnp.sum(deg_blk, axis=1) + 1.0
  return lax.rsqrt(d)


def _t1_scale(x, deg):
  def body(deg_ref, x_ref, y_ref):
    dinv = _deg_to_dinv(deg_ref[...])
    y_ref[0] = x_ref[...] * dinv[:, None]

  return pl.pallas_call(
      body,
      grid=(2, _GN),
      in_specs=[
          pl.BlockSpec((_NB, 32), lambda c, i: (i, 0)),
          pl.BlockSpec((_NB, 128), lambda c, i: (i, c)),
      ],
      out_specs=pl.BlockSpec((1, _NB, 128), lambda c, i: (c, i, 0)),
      out_shape=jax.ShapeDtypeStruct((2, _N, 128), jnp.float32),
  )(deg, x)


def _t2_conv1(x, s1, deg, w1, b1):
  def body(x_ref, s_ref, deg_ref, w_ref, b_ref, h_ref, st_ref):
    i = pl.program_id(0)
    dinv = _deg_to_dinv(deg_ref[...])
    di = dinv[:, None]
    di2 = (dinv * dinv)[:, None]
    u = jnp.concatenate(
        [s_ref[c] * di + x_ref[:, c * 128:(c + 1) * 128] * di2
         for c in range(2)], axis=1)
    h = jnp.dot(u, w_ref[...], preferred_element_type=jnp.float32) + b_ref[...]
    h_ref[...] = h

    @pl.when(i == 0)
    def _():
      st_ref[...] = jnp.zeros_like(st_ref)

    st_ref[...] += jnp.stack([h.sum(0), (h * h).sum(0)])

  return pl.pallas_call(
      body,
      grid=(_GN,),
      in_specs=[
          pl.BlockSpec((_NB, 256), lambda i: (i, 0)),
          pl.BlockSpec((2, _NB, 128), lambda i: (0, i, 0)),
          pl.BlockSpec((_NB, 32), lambda i: (i, 0)),
          pl.BlockSpec((256, 1024), lambda i: (0, 0)),
          pl.BlockSpec((1, 1024), lambda i: (0, 0)),
      ],
      out_specs=[
          pl.BlockSpec((_NB, 1024), lambda i: (i, 0)),
          pl.BlockSpec((2, 1024), lambda i: (0, 0)),
      ],
      out_shape=[
          jax.ShapeDtypeStruct((_N, 1024), jnp.float32),
          jax.ShapeDtypeStruct((2, 1024), jnp.float32),
      ],
  )(x, s1, deg, w1, b1)


def _bn_relu_score(h, stats, gamma, beta, pw, divisor):
  def body(h_ref, st_ref, g_ref, b_ref, p_ref, out_ref, sc_ref):
    mu = st_ref[0] * (1.0 / divisor)
    var = st_ref[1] * (1.0 / divisor) - mu * mu
    sc = lax.rsqrt(var + _EPS) * g_ref[0]
    sh = b_ref[0] - mu * sc
    g = jnp.maximum(h_ref[...] * sc[None, :] + sh[None, :], 0.0)
    out_ref[...] = g
    pwv = p_ref[...]
    nrm = jnp.sqrt(jnp.sum(pwv * pwv))
    sc_ref[...] = jnp.dot(g, pwv, preferred_element_type=jnp.float32) / nrm

  return pl.pallas_call(
      body,
      grid=(_GN,),
      in_specs=[
          pl.BlockSpec((_NB, 1024), lambda i: (i, 0)),
          pl.BlockSpec((2, 1024), lambda i: (0, 0)),
          pl.BlockSpec((1, 1024), lambda i: (0, 0)),
          pl.BlockSpec((1, 1024), lambda i: (0, 0)),
          pl.BlockSpec((1024, 1), lambda i: (0, 0)),
      ],
      out_specs=[
          pl.BlockSpec((_NB, 1024), lambda i: (i, 0)),
          pl.BlockSpec((_NB, 1), lambda i: (i, 0)),
      ],
      out_shape=[
          jax.ShapeDtypeStruct((_N, 1024), jnp.float32),
          jax.ShapeDtypeStruct((_N, 1), jnp.float32),
      ],
  )(h, stats, gamma, beta, pw)


def _topk_mask(score, elig, k):
  def body(s_ref, e_ref, keep_ref, w_ref):
    s = s_ref[...]
    e = e_ref[...] > 0.5
    bu = lax.bitcast_convert_type(s, jnp.uint32)
    neg = (bu >> jnp.uint32(31)) == jnp.uint32(1)
    key = bu ^ jnp.where(neg, jnp.uint32(0xFFFFFFFF), jnp.uint32(0x80000000))
    key = jnp.where(e, key, jnp.uint32(0))
    idx = lax.broadcasted_iota(jnp.int32, s.shape, 0)

    def bit_step(i, t):
      cand = t | (jnp.uint32(1) << (jnp.uint32(31) - i.astype(jnp.uint32)))
      cnt = jnp.sum((key >= cand).astype(jnp.int32))
      return jnp.where(cnt >= k, cand, t)

    thr = lax.fori_loop(0, 32, bit_step, jnp.uint32(0))
    ties = key == thr
    r = k - jnp.sum((key > thr).astype(jnp.int32))

    def j_step(i, jv):
      cand = jv | (jnp.int32(1) << (jnp.int32(13) - i))
      f = jnp.sum((ties & (idx < cand)).astype(jnp.int32))
      return jnp.where(f <= r, cand, jv)

    jcut = lax.fori_loop(0, 14, j_step, jnp.int32(0))
    keep = (key > thr) | (ties & (idx < jcut))
    keep_ref[...] = keep.astype(jnp.float32)
    w_ref[...] = jnp.where(keep, jnp.tanh(s), 0.0)

  return pl.pallas_call(
      body,
      out_shape=[
          jax.ShapeDtypeStruct((_N, 1), jnp.float32),
          jax.ShapeDtypeStruct((_N, 1), jnp.float32),
      ],
  )(score, elig)


def _t5_prep2(g1o, w1, deg2, k1):
  def body(g_ref, w_ref, deg_ref, y_ref, x1_ref):
    i = pl.program_id(1)
    dinv = _deg_to_dinv(deg_ref[...])
    wv = w_ref[:, 0]
    p = g_ref[...] * wv[:, None]

    @pl.when(i == 0)
    def _():
      x1_ref[...] = jnp.zeros_like(x1_ref)

    x1_ref[...] += p.sum(0, keepdims=True) * (1.0 / k1)
    y_ref[0] = p * dinv[:, None]

  return pl.pallas_call(
      body,
      grid=(8, _GN),
      in_specs=[
          pl.BlockSpec((_NB, 128), lambda c, i: (i, c)),
          pl.BlockSpec((_NB, 1), lambda c, i: (i, 0)),
          pl.BlockSpec((_NB, 32), lambda c, i: (i, 0)),
      ],
      out_specs=[
          pl.BlockSpec((1, _NB, 128), lambda c, i: (c, i, 0)),
          pl.BlockSpec((1, 128), lambda c, i: (0, c)),
      ],
      out_shape=[
          jax.ShapeDtypeStruct((8, _N, 128), jnp.float32),
          jax.ShapeDtypeStruct((1, 1024), jnp.float32),
      ],
  )(g1o, w1, deg2)


def _t6_conv2(s2, y2, deg2, keep1, w2, b2):
  def body(s_ref, y_ref, deg_ref, k_ref, w_ref, b_ref, h_ref, st_ref):
    i = pl.program_id(0)
    dinv = _deg_to_dinv(deg_ref[...])[:, None]
    u = jnp.concatenate(
        [(s_ref[c] + y_ref[c]) * dinv for c in range(8)], axis=1)
    h = jnp.dot(u, w_ref[...], preferred_element_type=jnp.float32) + b_ref[...]
    h_ref[...] = h
    hm = h * k_ref[...]

    @pl.when(i == 0)
    def _():
      st_ref[...] = jnp.zeros_like(st_ref)

    st_ref[...] += jnp.stack([hm.sum(0), (h * hm).sum(0)])

  return pl.pallas_call(
      body,
      grid=(_GN,),
      in_specs=[
          pl.BlockSpec((8, _NB, 128), lambda i: (0, i, 0)),
          pl.BlockSpec((8, _NB, 128), lambda i: (0, i, 0)),
          pl.BlockSpec((_NB, 32), lambda i: (i, 0)),
          pl.BlockSpec((_NB, 1), lambda i: (i, 0)),
          pl.BlockSpec((1024, 1024), lambda i: (0, 0)),
          pl.BlockSpec((1, 1024), lambda i: (0, 0)),
      ],
      out_specs=[
          pl.BlockSpec((_NB, 1024), lambda i: (i, 0)),
          pl.BlockSpec((2, 1024), lambda i: (0, 0)),
      ],
      out_shape=[
          jax.ShapeDtypeStruct((_N, 1024), jnp.float32),
          jax.ShapeDtypeStruct((2, 1024), jnp.float32),
      ],
  )(s2, y2, deg2, keep1, w2, b2)


def _t9_final(g2o, w2, x1, wf, bf, wo, bo, k2):
  def body(g_ref, w_ref, x1_ref, wf_ref, bf_ref, wo_ref, bo_ref,
           out_ref, acc):
    i = pl.program_id(0)

    @pl.when(i == 0)
    def _():
      acc[...] = jnp.zeros_like(acc)

    wv = w_ref[:, 0]
    acc[...] += (g_ref[...] * wv[:, None]).sum(0, keepdims=True) * (1.0 / k2)

    @pl.when(i == _GN - 1)
    def _():
      z = x1_ref[...] + acc[...]
      zf = jnp.maximum(
          jnp.dot(z, wf_ref[...], preferred_element_type=jnp.float32)
          + bf_ref[...], 0.0)
      out_ref[...] = (
          jnp.dot(zf, wo_ref[...], preferred_element_type=jnp.float32)
          + bo_ref[...])

  return pl.pallas_call(
      body,
      grid=(_GN,),
      in_specs=[
          pl.BlockSpec((_NB, 1024), lambda i: (i, 0)),
          pl.BlockSpec((_NB, 1), lambda i: (i, 0)),
          pl.BlockSpec((1, 1024), lambda i: (0, 0)),
          pl.BlockSpec((1024, 512), lambda i: (0, 0)),
          pl.BlockSpec((1, 512), lambda i: (0, 0)),
          pl.BlockSpec((512, 128), lambda i: (0, 0)),
          pl.BlockSpec((1, 128), lambda i: (0, 0)),
      ],
      out_specs=pl.BlockSpec((1, 128), lambda i: (0, 0)),
      out_shape=jax.ShapeDtypeStruct((1, 128), jnp.float32),
      scratch_shapes=[pltpu.VMEM((1, 1024), jnp.float32)],
  )(g2o, w2, x1, wf, bf, wo, bo)


def kernel(x, edge_index, batch, W1, b1, g1, bt1, pw1, W2, b2, g2, bt2, pw2,
           Wf, bf, Wo, bo):
  f32 = jnp.float32
  src = edge_index[0]
  dst = edge_index[1]
  npad = _EPAD - _E
  pad_dump = _N + (jnp.arange(npad, dtype=jnp.int32) % 16)
  src_p = jnp.concatenate(
      [src, jnp.zeros((npad,), jnp.int32)]).reshape(32, _NBM, _B)
  dst_p = jnp.concatenate([dst, pad_dump]).reshape(32, _NBM, _B)
  src16 = src_p.reshape(16, _NBR, _B)
  dst16 = dst_p.reshape(16, _NBR, _B)
  zeros_nr = jnp.zeros((1, _NR), f32)
  zeros128 = jnp.zeros((_RPT, _B), f32)
  ones_keep = jnp.ones((_NR,), f32)

  deg1 = _sc_deg(ones_keep, src_p, dst_p, zeros_nr)[:, 0, :].T
  y1 = _t1_scale(x, deg1)
  s1 = _sc_rows(y1, src16, dst16, zeros128, 2)
  h1, st1 = _t2_conv1(x, s1, deg1, W1, b1.reshape(1, -1))
  g1o, sc1 = _bn_relu_score(h1, st1, g1.reshape(1, -1), bt1.reshape(1, -1),
                            pw1.reshape(-1, 1), float(_N))
  keep1, w1 = _topk_mask(sc1, jnp.ones((_N, 1), f32), 5000)

  keep_ext = jnp.concatenate([keep1.reshape(-1), jnp.ones((_ND,), f32)])
  deg2 = _sc_deg(keep_ext, src_p, dst_p, zeros_nr)[:, 0, :].T
  y2, x1 = _t5_prep2(g1o, w1, deg2, 5000.0)
  s2 = _sc_rows(y2, src16, dst16, zeros128, 8)
  h2, st2 = _t6_conv2(s2, y2, deg2, keep1, W2, b2.reshape(1, -1))
  g2o, sc2 = _bn_relu_score(h2, st2, g2.reshape(1, -1), bt2.reshape(1, -1),
                            pw2.reshape(-1, 1), 5000.0)
  _, w2 = _topk_mask(sc2, keep1, 2500)

  return _t9_final(g2o, w2, x1, Wf, bf.reshape(1, -1), Wo, bo.reshape(1, -1),
                   2500.0)
